# initial kernel scaffold (unmeasured)
import jax
import jax.numpy as jnp
from jax import lax
from jax.experimental import pallas as pl
from jax.experimental.pallas import tpu as pltpu

T = 2048
D = 4096
V_LOCAL = 8192


def _exchange(logits):

    def body(lg_ref, out_ref, send_sem, recv_sem):
        my_x = lax.axis_index("x")
        my_y = lax.axis_index("y")
        my_z = lax.axis_index("z")
        partner = (my_x, 1 - my_y, my_z)

        barrier = pltpu.get_barrier_semaphore()
        pl.semaphore_signal(
            barrier, inc=1, device_id=partner,
            device_id_type=pl.DeviceIdType.MESH,
        )
        pl.semaphore_wait(barrier, 1)

        out_ref[:, pl.ds(my_y * V_LOCAL, V_LOCAL)] = lg_ref[...]
        rdma = pltpu.make_async_remote_copy(
            src_ref=lg_ref,
            dst_ref=out_ref.at[:, pl.ds(my_y * V_LOCAL, V_LOCAL)],
            send_sem=send_sem,
            recv_sem=recv_sem,
            device_id=partner,
            device_id_type=pl.DeviceIdType.MESH,
        )
        rdma.start()
        rdma.wait()

    return pl.pallas_call(
        body,
        out_shape=jax.ShapeDtypeStruct((T, 2 * V_LOCAL), jnp.bfloat16),
        in_specs=[pl.BlockSpec(memory_space=pltpu.VMEM)],
        out_specs=pl.BlockSpec(memory_space=pltpu.VMEM),
        scratch_shapes=[pltpu.SemaphoreType.DMA, pltpu.SemaphoreType.DMA],
        compiler_params=pltpu.CompilerParams(collective_id=0),
    )(logits)


def kernel(x, W):
    xb = x.astype(jnp.bfloat16)
    Wb = W.astype(jnp.bfloat16)
    logits = jnp.dot(
        xb, Wb, preferred_element_type=jnp.float32
    ).astype(jnp.bfloat16)
    full = _exchange(logits).astype(jnp.float32)
    m = jnp.max(full, axis=-1, keepdims=True)
    e = jnp.exp(full - m)
    return e / jnp.sum(e, axis=-1, keepdims=True)


# baseline (device time: 717460 ns/iter reference)
import jax
import jax.numpy as jnp
from jax import lax
from jax.experimental import pallas as pl
from jax.experimental.pallas import tpu as pltpu

T = 2048
D = 4096
V_LOCAL = 8192
TB = 128
N_BLK = T // TB


def _exchange_softmax(logits):

    def body(lg_ref, out_ref, comm_ref, send_sems, recv_sems):
        i = pl.program_id(0)
        my_x = lax.axis_index("x")
        my_y = lax.axis_index("y")
        my_z = lax.axis_index("z")
        partner = (my_x, 1 - my_y, my_z)
        slot = lax.rem(i, 2)

        @pl.when(i == 0)
        def _():
            barrier = pltpu.get_barrier_semaphore()
            pl.semaphore_signal(
                barrier, inc=1, device_id=partner,
                device_id_type=pl.DeviceIdType.MESH,
            )
            pl.semaphore_wait(barrier, 1)

        rdma = pltpu.make_async_remote_copy(
            src_ref=lg_ref,
            dst_ref=comm_ref.at[slot],
            send_sem=send_sems.at[slot],
            recv_sem=recv_sems.at[slot],
            device_id=partner,
            device_id_type=pl.DeviceIdType.MESH,
        )
        rdma.start()
        rdma.wait()

        mine = lg_ref[...].astype(jnp.float32)
        theirs = comm_ref[slot].astype(jnp.float32)
        m = jnp.maximum(
            jnp.max(mine, axis=-1, keepdims=True),
            jnp.max(theirs, axis=-1, keepdims=True),
        )
        e_mine = jnp.exp(mine - m)
        e_theirs = jnp.exp(theirs - m)
        denom = (
            jnp.sum(e_mine, axis=-1, keepdims=True)
            + jnp.sum(e_theirs, axis=-1, keepdims=True)
        )
        out_ref[:, pl.ds(my_y * V_LOCAL, V_LOCAL)] = e_mine / denom
        out_ref[:, pl.ds((1 - my_y) * V_LOCAL, V_LOCAL)] = e_theirs / denom

    return pl.pallas_call(
        body,
        grid=(N_BLK,),
        out_shape=jax.ShapeDtypeStruct((T, 2 * V_LOCAL), jnp.float32),
        in_specs=[
            pl.BlockSpec((TB, V_LOCAL), lambda i: (i, 0),
                         memory_space=pltpu.VMEM),
        ],
        out_specs=pl.BlockSpec((TB, 2 * V_LOCAL), lambda i: (i, 0),
                               memory_space=pltpu.VMEM),
        scratch_shapes=[
            pltpu.VMEM((2, TB, V_LOCAL), jnp.bfloat16),
            pltpu.SemaphoreType.DMA((2,)),
            pltpu.SemaphoreType.DMA((2,)),
        ],
        compiler_params=pltpu.CompilerParams(collective_id=0),
    )(logits)


def kernel(x, W):
    xb = x.astype(jnp.bfloat16)
    Wb = W.astype(jnp.bfloat16)
    logits = jnp.dot(
        xb, Wb, preferred_element_type=jnp.float32
    ).astype(jnp.bfloat16)
    return _exchange_softmax(logits)


# device time: 658986 ns/iter; 1.0887x vs baseline; 1.0887x over previous
import jax
import jax.numpy as jnp
from jax import lax
from jax.experimental import pallas as pl
from jax.experimental.pallas import tpu as pltpu

T = 2048
D = 4096
V_LOCAL = 8192
TB = 128
N_BLK = T // TB


def _exchange_softmax(logits):

    def body(lg_ref, out_hbm, comm, out_buf,
             send_sems, recv_sems, out_sems, credit_sems):
        my_x = lax.axis_index("x")
        my_y = lax.axis_index("y")
        my_z = lax.axis_index("z")
        partner = (my_x, 1 - my_y, my_z)

        def rdma_for(i):
            return pltpu.make_async_remote_copy(
                src_ref=lg_ref.at[pl.ds(i * TB, TB), :],
                dst_ref=comm.at[i % 2],
                send_sem=send_sems.at[i % 2],
                recv_sem=recv_sems.at[i % 2],
                device_id=partner,
                device_id_type=pl.DeviceIdType.MESH,
            )

        def out_copy_for(i):
            return pltpu.make_async_copy(
                out_buf.at[i % 2],
                out_hbm.at[pl.ds(i * TB, TB), :],
                out_sems.at[i % 2],
            )

        barrier = pltpu.get_barrier_semaphore()
        pl.semaphore_signal(
            barrier, inc=1, device_id=partner,
            device_id_type=pl.DeviceIdType.MESH,
        )
        pl.semaphore_wait(barrier, 1)

        rdma_for(0).start()

        for i in range(N_BLK):
            slot = i % 2
            if i + 1 < N_BLK:
                if i + 1 >= 2:
                    rdma_for(i - 1).wait_send()
                    pl.semaphore_wait(credit_sems.at[(i + 1) % 2], 1)
                rdma_for(i + 1).start()

            rdma_for(i).wait_recv()

            if i >= 2:
                out_copy_for(i - 2).wait()

            mine = lg_ref[pl.ds(i * TB, TB), :].astype(jnp.float32)
            theirs = comm[slot].astype(jnp.float32)
            m = jnp.maximum(
                jnp.max(mine, axis=-1, keepdims=True),
                jnp.max(theirs, axis=-1, keepdims=True),
            )
            e_mine = jnp.exp(mine - m)
            e_theirs = jnp.exp(theirs - m)
            denom = (
                jnp.sum(e_mine, axis=-1, keepdims=True)
                + jnp.sum(e_theirs, axis=-1, keepdims=True)
            )
            out_buf[slot, :, pl.ds(my_y * V_LOCAL, V_LOCAL)] = e_mine / denom
            out_buf[slot, :, pl.ds((1 - my_y) * V_LOCAL, V_LOCAL)] = (
                e_theirs / denom
            )
            if i + 2 < N_BLK:
                pl.semaphore_signal(
                    credit_sems.at[slot], inc=1, device_id=partner,
                    device_id_type=pl.DeviceIdType.MESH,
                )
            out_copy_for(i).start()

        rdma_for(N_BLK - 2).wait_send()
        rdma_for(N_BLK - 1).wait_send()
        out_copy_for(N_BLK - 2).wait()
        out_copy_for(N_BLK - 1).wait()

    return pl.pallas_call(
        body,
        out_shape=jax.ShapeDtypeStruct((T, 2 * V_LOCAL), jnp.float32),
        in_specs=[pl.BlockSpec(memory_space=pltpu.VMEM)],
        out_specs=pl.BlockSpec(memory_space=pl.ANY),
        scratch_shapes=[
            pltpu.VMEM((2, TB, V_LOCAL), jnp.bfloat16),
            pltpu.VMEM((2, TB, 2 * V_LOCAL), jnp.float32),
            pltpu.SemaphoreType.DMA((2,)),
            pltpu.SemaphoreType.DMA((2,)),
            pltpu.SemaphoreType.DMA((2,)),
            pltpu.SemaphoreType.REGULAR((2,)),
        ],
        compiler_params=pltpu.CompilerParams(
            collective_id=0, vmem_limit_bytes=63 * 1024 * 1024
        ),
    )(logits)


def kernel(x, W):
    xb = x.astype(jnp.bfloat16)
    Wb = W.astype(jnp.bfloat16)
    logits = jnp.dot(
        xb, Wb, preferred_element_type=jnp.float32
    ).astype(jnp.bfloat16)
    return _exchange_softmax(logits)


# device time: 573415 ns/iter; 1.2512x vs baseline; 1.1492x over previous
import jax
import jax.numpy as jnp
from jax import lax
from jax.experimental import pallas as pl
from jax.experimental.pallas import tpu as pltpu

T = 2048
D = 4096
V_LOCAL = 8192
TB = 256
NB = T // TB
SB = 128
WC = 512
NC = V_LOCAL // WC


def _fused(xb, Wb):

    def body(x_hbm, W_hbm, out_hbm, xbuf, wbuf, lg, comm, out_buf,
             xsems, wsems, send_sems, recv_sems, out_sems, credit_sems):
        my_x = lax.axis_index("x")
        my_y = lax.axis_index("y")
        my_z = lax.axis_index("z")
        partner = (my_x, 1 - my_y, my_z)

        def x_dma(b):
            return pltpu.make_async_copy(
                x_hbm.at[pl.ds(b * TB, TB), :],
                xbuf.at[b % 2],
                xsems.at[b % 2],
            )

        def rdma_for(b):
            return pltpu.make_async_remote_copy(
                src_ref=lg.at[b % 2],
                dst_ref=comm.at[b % 2],
                send_sem=send_sems.at[b % 2],
                recv_sem=recv_sems.at[b % 2],
                device_id=partner,
                device_id_type=pl.DeviceIdType.MESH,
            )

        def out_copy_for(b, h):
            return pltpu.make_async_copy(
                out_buf.at[h],
                out_hbm.at[pl.ds(b * TB + h * SB, SB), :],
                out_sems.at[h],
            )

        def gemm_block(b):
            slot = b % 2
            x_dma(b).wait()

            def wdma(c, s):
                return pltpu.make_async_copy(
                    W_hbm.at[:, pl.ds(c * WC, WC)], wbuf.at[s], wsems.at[s]
                )

            wdma(0, 0).start()

            def chunk(c, carry):
                cur = lax.rem(c, 2)
                nxt = 1 - cur

                @pl.when(c + 1 < NC)
                def _():
                    wdma(c + 1, nxt).start()

                wdma(c, cur).wait()
                part = jnp.dot(
                    xbuf[slot], wbuf[cur],
                    preferred_element_type=jnp.float32,
                )
                lg[slot, :, pl.ds(c * WC, WC)] = part.astype(jnp.bfloat16)
                return carry

            lax.fori_loop(0, NC, chunk, 0)

        barrier = pltpu.get_barrier_semaphore()
        pl.semaphore_signal(
            barrier, inc=1, device_id=partner,
            device_id_type=pl.DeviceIdType.MESH,
        )
        pl.semaphore_wait(barrier, 1)

        x_dma(0).start()
        gemm_block(0)
        rdma_for(0).start()

        for b in range(NB):
            slot = b % 2
            if b + 1 < NB:
                x_dma(b + 1).start()
                if b + 1 >= 2:
                    rdma_for(b - 1).wait_send()
                gemm_block(b + 1)
                if b + 1 >= 2:
                    pl.semaphore_wait(credit_sems.at[(b + 1) % 2], 1)
                rdma_for(b + 1).start()

            rdma_for(b).wait_recv()

            for h in range(2):
                if b >= 1:
                    out_copy_for(b - 1, h).wait()
                mine = lg[slot, pl.ds(h * SB, SB), :].astype(jnp.float32)
                theirs = comm[slot, pl.ds(h * SB, SB), :].astype(jnp.float32)
                m = jnp.maximum(
                    jnp.max(mine, axis=-1, keepdims=True),
                    jnp.max(theirs, axis=-1, keepdims=True),
                )
                e_mine = jnp.exp(mine - m)
                e_theirs = jnp.exp(theirs - m)
                denom = (
                    jnp.sum(e_mine, axis=-1, keepdims=True)
                    + jnp.sum(e_theirs, axis=-1, keepdims=True)
                )
                out_buf[h, :, pl.ds(my_y * V_LOCAL, V_LOCAL)] = e_mine / denom
                out_buf[h, :, pl.ds((1 - my_y) * V_LOCAL, V_LOCAL)] = (
                    e_theirs / denom
                )
                out_copy_for(b, h).start()

            if b + 2 < NB:
                pl.semaphore_signal(
                    credit_sems.at[slot], inc=1, device_id=partner,
                    device_id_type=pl.DeviceIdType.MESH,
                )

        rdma_for(NB - 2).wait_send()
        rdma_for(NB - 1).wait_send()
        out_copy_for(NB - 1, 0).wait()
        out_copy_for(NB - 1, 1).wait()

    return pl.pallas_call(
        body,
        out_shape=jax.ShapeDtypeStruct((T, 2 * V_LOCAL), jnp.float32),
        in_specs=[
            pl.BlockSpec(memory_space=pl.ANY),
            pl.BlockSpec(memory_space=pl.ANY),
        ],
        out_specs=pl.BlockSpec(memory_space=pl.ANY),
        scratch_shapes=[
            pltpu.VMEM((2, TB, D), jnp.bfloat16),
            pltpu.VMEM((2, D, WC), jnp.bfloat16),
            pltpu.VMEM((2, TB, V_LOCAL), jnp.bfloat16),
            pltpu.VMEM((2, TB, V_LOCAL), jnp.bfloat16),
            pltpu.VMEM((2, SB, 2 * V_LOCAL), jnp.float32),
            pltpu.SemaphoreType.DMA((2,)),
            pltpu.SemaphoreType.DMA((2,)),
            pltpu.SemaphoreType.DMA((2,)),
            pltpu.SemaphoreType.DMA((2,)),
            pltpu.SemaphoreType.DMA((2,)),
            pltpu.SemaphoreType.REGULAR((2,)),
        ],
        compiler_params=pltpu.CompilerParams(
            collective_id=0, vmem_limit_bytes=63 * 1024 * 1024
        ),
    )(xb, Wb)


def kernel(x, W):
    return _fused(x.astype(jnp.bfloat16), W.astype(jnp.bfloat16))


# device time: 529667 ns/iter; 1.3545x vs baseline; 1.0826x over previous
import jax
import jax.numpy as jnp
from jax import lax
from jax.experimental import pallas as pl
from jax.experimental.pallas import tpu as pltpu

T = 2048
D = 4096
V_LOCAL = 8192
TB = 256
NB = T // TB
SB = 128
WC = 512
NC = V_LOCAL // WC


def _fused(xb, Wb):

    def body(x_hbm, W_hbm, out_hbm, xbuf, wbuf, lg, comm, out_buf,
             xsems, wsems, send_sems, recv_sems, out_sems, credit_sems):
        my_x = lax.axis_index("x")
        my_y = lax.axis_index("y")
        my_z = lax.axis_index("z")
        partner = (my_x, 1 - my_y, my_z)

        def x_dma(b):
            return pltpu.make_async_copy(
                x_hbm.at[pl.ds(b * TB, TB), :],
                xbuf.at[b % 2],
                xsems.at[b % 2],
            )

        def rdma_for(b):
            return pltpu.make_async_remote_copy(
                src_ref=lg.at[b % 2],
                dst_ref=comm.at[b % 2],
                send_sem=send_sems.at[b % 2],
                recv_sem=recv_sems.at[b % 2],
                device_id=partner,
                device_id_type=pl.DeviceIdType.MESH,
            )

        def out_copy_for(b, h):
            return pltpu.make_async_copy(
                out_buf.at[h],
                out_hbm.at[pl.ds(b * TB + h * SB, SB), :],
                out_sems.at[h],
            )

        def gemm_block(b):
            slot = b % 2
            x_dma(b).wait()

            def wdma(c, s):
                return pltpu.make_async_copy(
                    W_hbm.at[:, pl.ds(c * WC, WC)], wbuf.at[s], wsems.at[s]
                )

            wdma(0, 0).start()

            def chunk(c, carry):
                cur = lax.rem(c, 2)
                nxt = 1 - cur

                @pl.when(c + 1 < NC)
                def _():
                    wdma(c + 1, nxt).start()

                wdma(c, cur).wait()
                part = jnp.dot(
                    xbuf[slot], wbuf[cur],
                    preferred_element_type=jnp.float32,
                )
                lg[slot, :, pl.ds(c * WC, WC)] = part.astype(jnp.bfloat16)
                return carry

            lax.fori_loop(0, NC, chunk, 0)

        barrier = pltpu.get_barrier_semaphore()
        pl.semaphore_signal(
            barrier, inc=1, device_id=partner,
            device_id_type=pl.DeviceIdType.MESH,
        )
        pl.semaphore_wait(barrier, 1)

        x_dma(0).start()
        gemm_block(0)
        rdma_for(0).start()

        for b in range(NB):
            slot = b % 2
            if b + 1 < NB:
                x_dma(b + 1).start()
                if b + 1 >= 2:
                    rdma_for(b - 1).wait_send()
                gemm_block(b + 1)
                if b + 1 >= 2:
                    pl.semaphore_wait(credit_sems.at[(b + 1) % 2], 1)
                rdma_for(b + 1).start()

            rdma_for(b).wait_recv()

            for h in range(2):
                if b >= 1:
                    out_copy_for(b - 1, h).wait()
                mine = lg[slot, pl.ds(h * SB, SB), :].astype(jnp.float32)
                theirs = comm[slot, pl.ds(h * SB, SB), :].astype(jnp.float32)
                m = jnp.maximum(
                    jnp.max(mine, axis=-1, keepdims=True),
                    jnp.max(theirs, axis=-1, keepdims=True),
                )
                e_mine = jnp.exp(mine - m)
                e_theirs = jnp.exp(theirs - m)
                denom = (
                    jnp.sum(e_mine, axis=-1, keepdims=True)
                    + jnp.sum(e_theirs, axis=-1, keepdims=True)
                )
                out_buf[h, :, pl.ds(my_y * V_LOCAL, V_LOCAL)] = (
                    e_mine / denom
                ).astype(jnp.bfloat16)
                out_buf[h, :, pl.ds((1 - my_y) * V_LOCAL, V_LOCAL)] = (
                    e_theirs / denom
                ).astype(jnp.bfloat16)
                out_copy_for(b, h).start()

            if b + 2 < NB:
                pl.semaphore_signal(
                    credit_sems.at[slot], inc=1, device_id=partner,
                    device_id_type=pl.DeviceIdType.MESH,
                )

        rdma_for(NB - 2).wait_send()
        rdma_for(NB - 1).wait_send()
        out_copy_for(NB - 1, 0).wait()
        out_copy_for(NB - 1, 1).wait()

    return pl.pallas_call(
        body,
        out_shape=jax.ShapeDtypeStruct((T, 2 * V_LOCAL), jnp.bfloat16),
        in_specs=[
            pl.BlockSpec(memory_space=pl.ANY),
            pl.BlockSpec(memory_space=pl.ANY),
        ],
        out_specs=pl.BlockSpec(memory_space=pl.ANY),
        scratch_shapes=[
            pltpu.VMEM((2, TB, D), jnp.bfloat16),
            pltpu.VMEM((2, D, WC), jnp.bfloat16),
            pltpu.VMEM((2, TB, V_LOCAL), jnp.bfloat16),
            pltpu.VMEM((2, TB, V_LOCAL), jnp.bfloat16),
            pltpu.VMEM((2, SB, 2 * V_LOCAL), jnp.bfloat16),
            pltpu.SemaphoreType.DMA((2,)),
            pltpu.SemaphoreType.DMA((2,)),
            pltpu.SemaphoreType.DMA((2,)),
            pltpu.SemaphoreType.DMA((2,)),
            pltpu.SemaphoreType.DMA((2,)),
            pltpu.SemaphoreType.REGULAR((2,)),
        ],
        compiler_params=pltpu.CompilerParams(
            collective_id=0, vmem_limit_bytes=63 * 1024 * 1024
        ),
    )(xb, Wb)


def kernel(x, W):
    return _fused(x.astype(jnp.bfloat16), W.astype(jnp.bfloat16))


# device time: 517178 ns/iter; 1.3873x vs baseline; 1.0241x over previous
import jax
import jax.numpy as jnp
from jax import lax
from jax.experimental import pallas as pl
from jax.experimental.pallas import tpu as pltpu

T = 2048
D = 4096
V_LOCAL = 8192
TB = 256
NB = T // TB
SB = 128
WC = 256
NC = V_LOCAL // WC


def _fused(x, W):

    def body(x_hbm, W_hbm, out_hbm, wb16_hbm,
             xf32, xbf, wf32, wbuf, lg, comm, out_buf,
             xsems, wfsems, wsems, wbsems,
             send_sems, recv_sems, out_sems, credit_sems):
        my_x = lax.axis_index("x")
        my_y = lax.axis_index("y")
        my_z = lax.axis_index("z")
        partner = (my_x, 1 - my_y, my_z)

        def x_dma(b):
            return pltpu.make_async_copy(
                x_hbm.at[pl.ds(b * TB, TB), :],
                xf32.at[b % 2],
                xsems.at[b % 2],
            )

        def rdma_for(b):
            return pltpu.make_async_remote_copy(
                src_ref=lg.at[b % 2],
                dst_ref=comm.at[b % 2],
                send_sem=send_sems.at[b % 2],
                recv_sem=recv_sems.at[b % 2],
                device_id=partner,
                device_id_type=pl.DeviceIdType.MESH,
            )

        def out_copy_for(b, h):
            return pltpu.make_async_copy(
                out_buf.at[h],
                out_hbm.at[pl.ds(b * TB + h * SB, SB), :],
                out_sems.at[h],
            )

        def gemm_block0():
            x_dma(0).wait()
            xbf[...] = xf32[0].astype(jnp.bfloat16)

            def wf_dma(c, s):
                return pltpu.make_async_copy(
                    W_hbm.at[:, pl.ds(c * WC, WC)], wf32.at[s], wfsems.at[s]
                )

            def wb_dma(c, s):
                return pltpu.make_async_copy(
                    wbuf.at[s], wb16_hbm.at[c], wbsems.at[s]
                )

            wf_dma(0, 0).start()

            def chunk(c, carry):
                cur = lax.rem(c, 2)
                nxt = 1 - cur

                @pl.when(c + 1 < NC)
                def _():
                    wf_dma(c + 1, nxt).start()

                wf_dma(c, cur).wait()

                @pl.when(c >= 2)
                def _():
                    wb_dma(c - 2, cur).wait()

                wbuf[cur] = wf32[cur].astype(jnp.bfloat16)
                part = jnp.dot(
                    xbf[...], wbuf[cur],
                    preferred_element_type=jnp.float32,
                )
                lg[0, :, pl.ds(c * WC, WC)] = part.astype(jnp.bfloat16)
                wb_dma(c, cur).start()
                return carry

            lax.fori_loop(0, NC, chunk, 0)
            wb_dma(NC - 2, 0).wait()
            wb_dma(NC - 1, 1).wait()

        def gemm_block(b):
            slot = b % 2
            x_dma(b).wait()
            xbf[...] = xf32[slot].astype(jnp.bfloat16)

            def wdma(c, s):
                return pltpu.make_async_copy(
                    wb16_hbm.at[c], wbuf.at[s], wsems.at[s]
                )

            wdma(0, 0).start()

            def chunk(c, carry):
                cur = lax.rem(c, 2)
                nxt = 1 - cur

                @pl.when(c + 1 < NC)
                def _():
                    wdma(c + 1, nxt).start()

                wdma(c, cur).wait()
                part = jnp.dot(
                    xbf[...], wbuf[cur],
                    preferred_element_type=jnp.float32,
                )
                lg[slot, :, pl.ds(c * WC, WC)] = part.astype(jnp.bfloat16)
                return carry

            lax.fori_loop(0, NC, chunk, 0)

        barrier = pltpu.get_barrier_semaphore()
        pl.semaphore_signal(
            barrier, inc=1, device_id=partner,
            device_id_type=pl.DeviceIdType.MESH,
        )
        pl.semaphore_wait(barrier, 1)

        x_dma(0).start()
        gemm_block0()
        rdma_for(0).start()

        for b in range(NB):
            slot = b % 2
            if b + 1 < NB:
                x_dma(b + 1).start()
                if b + 1 >= 2:
                    rdma_for(b - 1).wait_send()
                gemm_block(b + 1)
                if b + 1 >= 2:
                    pl.semaphore_wait(credit_sems.at[(b + 1) % 2], 1)
                rdma_for(b + 1).start()

            rdma_for(b).wait_recv()

            for h in range(2):
                if b >= 1:
                    out_copy_for(b - 1, h).wait()
                mine = lg[slot, pl.ds(h * SB, SB), :].astype(jnp.float32)
                theirs = comm[slot, pl.ds(h * SB, SB), :].astype(jnp.float32)
                m = jnp.maximum(
                    jnp.max(mine, axis=-1, keepdims=True),
                    jnp.max(theirs, axis=-1, keepdims=True),
                )
                e_mine = jnp.exp(mine - m)
                e_theirs = jnp.exp(theirs - m)
                denom = (
                    jnp.sum(e_mine, axis=-1, keepdims=True)
                    + jnp.sum(e_theirs, axis=-1, keepdims=True)
                )
                out_buf[h, :, pl.ds(my_y * V_LOCAL, V_LOCAL)] = (
                    e_mine / denom
                ).astype(jnp.bfloat16)
                out_buf[h, :, pl.ds((1 - my_y) * V_LOCAL, V_LOCAL)] = (
                    e_theirs / denom
                ).astype(jnp.bfloat16)
                out_copy_for(b, h).start()

            if b + 2 < NB:
                pl.semaphore_signal(
                    credit_sems.at[slot], inc=1, device_id=partner,
                    device_id_type=pl.DeviceIdType.MESH,
                )

        rdma_for(NB - 2).wait_send()
        rdma_for(NB - 1).wait_send()
        out_copy_for(NB - 1, 0).wait()
        out_copy_for(NB - 1, 1).wait()

    out, _ = pl.pallas_call(
        body,
        out_shape=(
            jax.ShapeDtypeStruct((T, 2 * V_LOCAL), jnp.bfloat16),
            jax.ShapeDtypeStruct((NC, D, WC), jnp.bfloat16),
        ),
        in_specs=[
            pl.BlockSpec(memory_space=pl.ANY),
            pl.BlockSpec(memory_space=pl.ANY),
        ],
        out_specs=(
            pl.BlockSpec(memory_space=pl.ANY),
            pl.BlockSpec(memory_space=pl.ANY),
        ),
        scratch_shapes=[
            pltpu.VMEM((2, TB, D), jnp.float32),
            pltpu.VMEM((TB, D), jnp.bfloat16),
            pltpu.VMEM((2, D, WC), jnp.float32),
            pltpu.VMEM((2, D, WC), jnp.bfloat16),
            pltpu.VMEM((2, TB, V_LOCAL), jnp.bfloat16),
            pltpu.VMEM((2, TB, V_LOCAL), jnp.bfloat16),
            pltpu.VMEM((2, SB, 2 * V_LOCAL), jnp.bfloat16),
            pltpu.SemaphoreType.DMA((2,)),
            pltpu.SemaphoreType.DMA((2,)),
            pltpu.SemaphoreType.DMA((2,)),
            pltpu.SemaphoreType.DMA((2,)),
            pltpu.SemaphoreType.DMA((2,)),
            pltpu.SemaphoreType.DMA((2,)),
            pltpu.SemaphoreType.DMA((2,)),
            pltpu.SemaphoreType.REGULAR((2,)),
        ],
        compiler_params=pltpu.CompilerParams(
            collective_id=0, vmem_limit_bytes=63 * 1024 * 1024
        ),
    )(x, W)
    return out


def kernel(x, W):
    return _fused(x, W)


# device time: 471708 ns/iter; 1.5210x vs baseline; 1.0964x over previous
import jax
import jax.numpy as jnp
from jax import lax
from jax.experimental import pallas as pl
from jax.experimental.pallas import tpu as pltpu

T = 2048
D = 4096
V_LOCAL = 8192
TB = 256
NB = T // TB
SB = 64
WC = 512
NC = V_LOCAL // WC
VH = V_LOCAL // 2


def _fused(x, W):

    def body(x_hbm, W_hbm, out_hbm, wb16_hbm,
             xf32, xbf, wf32, wbuf, lg, comm, out_buf,
             xsems, wfsems, wsems, wbsems,
             send_sems, recv_sems, h_send_sems, h_recv_sems,
             out_sem, credit_sems):
        my_x = lax.axis_index("x")
        my_y = lax.axis_index("y")
        my_z = lax.axis_index("z")
        partner = (my_x, 1 - my_y, my_z)

        def x_dma(b):
            return pltpu.make_async_copy(
                x_hbm.at[pl.ds(b * TB, TB), :], xf32, xsems,
            )

        def rdma_for(b):
            return pltpu.make_async_remote_copy(
                src_ref=lg.at[b % 2],
                dst_ref=comm.at[b % 2],
                send_sem=send_sems.at[b % 2],
                recv_sem=recv_sems.at[b % 2],
                device_id=partner,
                device_id_type=pl.DeviceIdType.MESH,
            )

        def rdma_half(h):
            return pltpu.make_async_remote_copy(
                src_ref=lg.at[0, :, pl.ds(h * VH, VH)],
                dst_ref=comm.at[0, :, pl.ds(h * VH, VH)],
                send_sem=h_send_sems.at[h],
                recv_sem=h_recv_sems.at[h],
                device_id=partner,
                device_id_type=pl.DeviceIdType.MESH,
            )

        def out_copy_for(b, h):
            return pltpu.make_async_copy(
                out_buf,
                out_hbm.at[pl.ds(b * TB + h * SB, SB), :],
                out_sem,
            )

        def gemm_block0():
            x_dma(0).wait()
            xbf[...] = xf32[...].astype(jnp.bfloat16)

            def wf_dma(c, s):
                return pltpu.make_async_copy(
                    W_hbm.at[:, pl.ds(c * WC, WC)], wf32.at[s], wfsems.at[s]
                )

            def wb_dma(c, s):
                return pltpu.make_async_copy(
                    wbuf.at[s], wb16_hbm.at[c], wbsems.at[s]
                )

            wf_dma(0, 0).start()

            def chunk(c, carry):
                cur = lax.rem(c, 2)
                nxt = 1 - cur

                @pl.when(c + 1 < NC)
                def _():
                    wf_dma(c + 1, nxt).start()

                wf_dma(c, cur).wait()

                @pl.when(c >= 2)
                def _():
                    wb_dma(c - 2, cur).wait()

                wbuf[cur] = wf32[cur].astype(jnp.bfloat16)
                part = jnp.dot(
                    xbf[...], wbuf[cur],
                    preferred_element_type=jnp.float32,
                )
                lg[0, :, pl.ds(c * WC, WC)] = part.astype(jnp.bfloat16)
                wb_dma(c, cur).start()

                @pl.when(c == NC // 2 - 1)
                def _():
                    rdma_half(0).start()

                return carry

            lax.fori_loop(0, NC, chunk, 0)
            wb_dma(NC - 2, 0).wait()
            wb_dma(NC - 1, 1).wait()
            rdma_half(1).start()

        def gemm_block(b):
            slot = b % 2
            x_dma(b).wait()
            xbf[...] = xf32[...].astype(jnp.bfloat16)

            def wdma(c, s):
                return pltpu.make_async_copy(
                    wb16_hbm.at[c], wbuf.at[s], wsems.at[s]
                )

            wdma(0, 0).start()

            def chunk(c, carry):
                cur = lax.rem(c, 2)
                nxt = 1 - cur

                @pl.when(c + 1 < NC)
                def _():
                    wdma(c + 1, nxt).start()

                wdma(c, cur).wait()
                part = jnp.dot(
                    xbf[...], wbuf[cur],
                    preferred_element_type=jnp.float32,
                )
                lg[slot, :, pl.ds(c * WC, WC)] = part.astype(jnp.bfloat16)
                return carry

            lax.fori_loop(0, NC, chunk, 0)

        barrier = pltpu.get_barrier_semaphore()
        pl.semaphore_signal(
            barrier, inc=1, device_id=partner,
            device_id_type=pl.DeviceIdType.MESH,
        )
        pl.semaphore_wait(barrier, 1)

        x_dma(0).start()
        gemm_block0()

        for b in range(NB):
            slot = b % 2
            if b + 1 < NB:
                x_dma(b + 1).start()
                if b + 1 >= 2:
                    if b - 1 == 0:
                        rdma_half(0).wait_send()
                        rdma_half(1).wait_send()
                    else:
                        rdma_for(b - 1).wait_send()
                gemm_block(b + 1)
                if b + 1 >= 2:
                    pl.semaphore_wait(credit_sems.at[(b + 1) % 2], 1)
                rdma_for(b + 1).start()

            if b == 0:
                rdma_half(0).wait_recv()
                rdma_half(1).wait_recv()
            else:
                rdma_for(b).wait_recv()

            for h in range(TB // SB):
                mine = lg[slot, pl.ds(h * SB, SB), :].astype(jnp.float32)
                theirs = comm[slot, pl.ds(h * SB, SB), :].astype(jnp.float32)
                m = jnp.maximum(
                    jnp.max(mine, axis=-1, keepdims=True),
                    jnp.max(theirs, axis=-1, keepdims=True),
                )
                e_mine = jnp.exp(mine - m)
                e_theirs = jnp.exp(theirs - m)
                denom = (
                    jnp.sum(e_mine, axis=-1, keepdims=True)
                    + jnp.sum(e_theirs, axis=-1, keepdims=True)
                )
                p_mine = (e_mine / denom).astype(jnp.bfloat16)
                p_theirs = (e_theirs / denom).astype(jnp.bfloat16)
                if b > 0 or h > 0:
                    out_copy_for(0, 0).wait()
                out_buf[:, pl.ds(my_y * V_LOCAL, V_LOCAL)] = p_mine
                out_buf[:, pl.ds((1 - my_y) * V_LOCAL, V_LOCAL)] = p_theirs
                out_copy_for(b, h).start()

            if b + 2 < NB:
                pl.semaphore_signal(
                    credit_sems.at[slot], inc=1, device_id=partner,
                    device_id_type=pl.DeviceIdType.MESH,
                )

        rdma_for(NB - 2).wait_send()
        rdma_for(NB - 1).wait_send()
        out_copy_for(0, 0).wait()

    out, _ = pl.pallas_call(
        body,
        out_shape=(
            jax.ShapeDtypeStruct((T, 2 * V_LOCAL), jnp.bfloat16),
            jax.ShapeDtypeStruct((NC, D, WC), jnp.bfloat16),
        ),
        in_specs=[
            pl.BlockSpec(memory_space=pl.ANY),
            pl.BlockSpec(memory_space=pl.ANY),
        ],
        out_specs=(
            pl.BlockSpec(memory_space=pl.ANY),
            pl.BlockSpec(memory_space=pl.ANY),
        ),
        scratch_shapes=[
            pltpu.VMEM((TB, D), jnp.float32),
            pltpu.VMEM((TB, D), jnp.bfloat16),
            pltpu.VMEM((2, D, WC), jnp.float32),
            pltpu.VMEM((2, D, WC), jnp.bfloat16),
            pltpu.VMEM((2, TB, V_LOCAL), jnp.bfloat16),
            pltpu.VMEM((2, TB, V_LOCAL), jnp.bfloat16),
            pltpu.VMEM((SB, 2 * V_LOCAL), jnp.bfloat16),
            pltpu.SemaphoreType.DMA,
            pltpu.SemaphoreType.DMA((2,)),
            pltpu.SemaphoreType.DMA((2,)),
            pltpu.SemaphoreType.DMA((2,)),
            pltpu.SemaphoreType.DMA((2,)),
            pltpu.SemaphoreType.DMA((2,)),
            pltpu.SemaphoreType.DMA((2,)),
            pltpu.SemaphoreType.DMA((2,)),
            pltpu.SemaphoreType.DMA,
            pltpu.SemaphoreType.REGULAR((2,)),
        ],
        compiler_params=pltpu.CompilerParams(
            collective_id=0, vmem_limit_bytes=63 * 1024 * 1024
        ),
    )(x, W)
    return out


def kernel(x, W):
    return _fused(x, W)


# device time: 465844 ns/iter; 1.5401x vs baseline; 1.0126x over previous
import jax
import jax.numpy as jnp
from jax import lax
from jax.experimental import pallas as pl
from jax.experimental.pallas import tpu as pltpu

T = 2048
D = 4096
V_LOCAL = 8192
TB = 256
NB = T // TB
SB = 64
WC = 512
NC = V_LOCAL // WC
VQ = V_LOCAL // 4
RH = TB // 2


def _fused(x, W):

    def body(x_hbm, W_hbm, out_hbm, wb16_hbm,
             xf32, xbf, wf32, wbuf, lg, comm, out_buf,
             xsems, wfsems, wsems, wbsems,
             send_sems, recv_sems, h_send_sems, h_recv_sems,
             r_send_sems, r_recv_sems, out_sem, credit_sems):
        my_x = lax.axis_index("x")
        my_y = lax.axis_index("y")
        my_z = lax.axis_index("z")
        partner = (my_x, 1 - my_y, my_z)

        def x_dma(b):
            return pltpu.make_async_copy(
                x_hbm.at[pl.ds(b * TB, TB), :], xf32, xsems,
            )

        def rdma_for(b):
            return pltpu.make_async_remote_copy(
                src_ref=lg.at[b % 2],
                dst_ref=comm.at[b % 2],
                send_sem=send_sems.at[b % 2],
                recv_sem=recv_sems.at[b % 2],
                device_id=partner,
                device_id_type=pl.DeviceIdType.MESH,
            )

        def rdma_q(q):
            return pltpu.make_async_remote_copy(
                src_ref=lg.at[0, :, pl.ds(q * VQ, VQ)],
                dst_ref=comm.at[0, :, pl.ds(q * VQ, VQ)],
                send_sem=h_send_sems.at[q],
                recv_sem=h_recv_sems.at[q],
                device_id=partner,
                device_id_type=pl.DeviceIdType.MESH,
            )

        def rdma_last(r):
            return pltpu.make_async_remote_copy(
                src_ref=lg.at[(NB - 1) % 2, pl.ds(r * RH, RH), :],
                dst_ref=comm.at[(NB - 1) % 2, pl.ds(r * RH, RH), :],
                send_sem=r_send_sems.at[r],
                recv_sem=r_recv_sems.at[r],
                device_id=partner,
                device_id_type=pl.DeviceIdType.MESH,
            )

        def out_copy_for(b, h):
            return pltpu.make_async_copy(
                out_buf,
                out_hbm.at[pl.ds(b * TB + h * SB, SB), :],
                out_sem,
            )

        def gemm_block0():
            x_dma(0).wait()
            xbf[...] = xf32[...].astype(jnp.bfloat16)

            def wf_dma(c, s):
                return pltpu.make_async_copy(
                    W_hbm.at[:, pl.ds(c * WC, WC)], wf32.at[s], wfsems.at[s]
                )

            def wb_dma(c, s):
                return pltpu.make_async_copy(
                    wbuf.at[s], wb16_hbm.at[c], wbsems.at[s]
                )

            wf_dma(0, 0).start()

            def chunk(c, carry):
                cur = lax.rem(c, 2)
                nxt = 1 - cur

                @pl.when(c + 1 < NC)
                def _():
                    wf_dma(c + 1, nxt).start()

                wf_dma(c, cur).wait()

                @pl.when(c >= 2)
                def _():
                    wb_dma(c - 2, cur).wait()

                wbuf[cur] = wf32[cur].astype(jnp.bfloat16)
                part = jnp.dot(
                    xbf[...], wbuf[cur],
                    preferred_element_type=jnp.float32,
                )
                lg[0, :, pl.ds(c * WC, WC)] = part.astype(jnp.bfloat16)
                wb_dma(c, cur).start()

                for q in range(3):
                    @pl.when(c == (q + 1) * (NC // 4) - 1)
                    def _(q=q):
                        rdma_q(q).start()

                return carry

            lax.fori_loop(0, NC, chunk, 0)
            wb_dma(NC - 2, 0).wait()
            wb_dma(NC - 1, 1).wait()
            rdma_q(3).start()

        def gemm_block(b):
            slot = b % 2
            x_dma(b).wait()
            xbf[...] = xf32[...].astype(jnp.bfloat16)

            def wdma(c, s):
                return pltpu.make_async_copy(
                    wb16_hbm.at[c], wbuf.at[s], wsems.at[s]
                )

            wdma(0, 0).start()

            def chunk(c, carry):
                cur = lax.rem(c, 2)
                nxt = 1 - cur

                @pl.when(c + 1 < NC)
                def _():
                    wdma(c + 1, nxt).start()

                wdma(c, cur).wait()
                part = jnp.dot(
                    xbf[...], wbuf[cur],
                    preferred_element_type=jnp.float32,
                )
                lg[slot, :, pl.ds(c * WC, WC)] = part.astype(jnp.bfloat16)
                return carry

            lax.fori_loop(0, NC, chunk, 0)

        barrier = pltpu.get_barrier_semaphore()
        pl.semaphore_signal(
            barrier, inc=1, device_id=partner,
            device_id_type=pl.DeviceIdType.MESH,
        )
        pl.semaphore_wait(barrier, 1)

        x_dma(0).start()
        gemm_block0()

        for b in range(NB):
            slot = b % 2
            if b + 1 < NB:
                x_dma(b + 1).start()
                if b + 1 >= 2:
                    if b - 1 == 0:
                        for q in range(4):
                            rdma_q(q).wait_send()
                    else:
                        rdma_for(b - 1).wait_send()
                gemm_block(b + 1)
                if b + 1 >= 2:
                    pl.semaphore_wait(credit_sems.at[(b + 1) % 2], 1)
                if b + 1 == NB - 1:
                    rdma_last(0).start()
                    rdma_last(1).start()
                else:
                    rdma_for(b + 1).start()

            if b == 0:
                for q in range(4):
                    rdma_q(q).wait_recv()
            elif b == NB - 1:
                rdma_last(0).wait_recv()
            else:
                rdma_for(b).wait_recv()

            for h in range(TB // SB):
                if b == NB - 1 and h == (TB // SB) // 2:
                    rdma_last(1).wait_recv()
                mine = lg[slot, pl.ds(h * SB, SB), :].astype(jnp.float32)
                theirs = comm[slot, pl.ds(h * SB, SB), :].astype(jnp.float32)
                m = jnp.maximum(
                    jnp.max(mine, axis=-1, keepdims=True),
                    jnp.max(theirs, axis=-1, keepdims=True),
                )
                e_mine = jnp.exp(mine - m)
                e_theirs = jnp.exp(theirs - m)
                denom = (
                    jnp.sum(e_mine, axis=-1, keepdims=True)
                    + jnp.sum(e_theirs, axis=-1, keepdims=True)
                )
                p_mine = (e_mine / denom).astype(jnp.bfloat16)
                p_theirs = (e_theirs / denom).astype(jnp.bfloat16)
                if b > 0 or h > 0:
                    out_copy_for(0, 0).wait()
                out_buf[:, pl.ds(my_y * V_LOCAL, V_LOCAL)] = p_mine
                out_buf[:, pl.ds((1 - my_y) * V_LOCAL, V_LOCAL)] = p_theirs
                out_copy_for(b, h).start()

            if b + 2 < NB:
                pl.semaphore_signal(
                    credit_sems.at[slot], inc=1, device_id=partner,
                    device_id_type=pl.DeviceIdType.MESH,
                )

        rdma_for(NB - 2).wait_send()
        rdma_last(0).wait_send()
        rdma_last(1).wait_send()
        out_copy_for(0, 0).wait()

    out, _ = pl.pallas_call(
        body,
        out_shape=(
            jax.ShapeDtypeStruct((T, 2 * V_LOCAL), jnp.bfloat16),
            jax.ShapeDtypeStruct((NC, D, WC), jnp.bfloat16),
        ),
        in_specs=[
            pl.BlockSpec(memory_space=pl.ANY),
            pl.BlockSpec(memory_space=pl.ANY),
        ],
        out_specs=(
            pl.BlockSpec(memory_space=pl.ANY),
            pl.BlockSpec(memory_space=pl.ANY),
        ),
        scratch_shapes=[
            pltpu.VMEM((TB, D), jnp.float32),
            pltpu.VMEM((TB, D), jnp.bfloat16),
            pltpu.VMEM((2, D, WC), jnp.float32),
            pltpu.VMEM((2, D, WC), jnp.bfloat16),
            pltpu.VMEM((2, TB, V_LOCAL), jnp.bfloat16),
            pltpu.VMEM((2, TB, V_LOCAL), jnp.bfloat16),
            pltpu.VMEM((SB, 2 * V_LOCAL), jnp.bfloat16),
            pltpu.SemaphoreType.DMA,
            pltpu.SemaphoreType.DMA((2,)),
            pltpu.SemaphoreType.DMA((2,)),
            pltpu.SemaphoreType.DMA((2,)),
            pltpu.SemaphoreType.DMA((2,)),
            pltpu.SemaphoreType.DMA((2,)),
            pltpu.SemaphoreType.DMA((4,)),
            pltpu.SemaphoreType.DMA((4,)),
            pltpu.SemaphoreType.DMA((2,)),
            pltpu.SemaphoreType.DMA((2,)),
            pltpu.SemaphoreType.DMA,
            pltpu.SemaphoreType.REGULAR((2,)),
        ],
        compiler_params=pltpu.CompilerParams(
            collective_id=0, vmem_limit_bytes=63 * 1024 * 1024
        ),
    )(x, W)
    return out


def kernel(x, W):
    return _fused(x, W)
